# Initial kernel scaffold; baseline (speedup 1.0000x reference)
import jax
import jax.numpy as jnp
from jax.experimental import pallas as pl


def kernel(logits, targets, puzzles):
    raise NotImplementedError("write your pallas kernel here")


# speedup vs baseline: 6.6448x; 6.6448x over previous
import jax
import jax.numpy as jnp
from jax.experimental import pallas as pl
from jax.experimental.pallas import tpu as pltpu

_CONSTRAINT_WEIGHT = 0.5
_FOCAL_GAMMA = 2.0
_EPS = 1e-8
_BC = 256


def _build_sel():
    ci = jnp.arange(27)[:, None]
    cell = jnp.arange(81)[None, :]
    r = cell // 9
    c = cell % 9
    bx = (r // 3) * 3 + (c // 3)
    sel = jnp.where(ci < 9, r == ci,
                    jnp.where(ci < 18, c == ci - 9, bx == ci - 18))
    return sel.astype(jnp.float32)


def _body(lt_ref, tg_ref, pz_ref, s_ref, out_ref):
    x = lt_ref[...]
    tgt = jnp.clip(tg_ref[...] - 1, 0, 8)
    mask = (pz_ref[...] == 0).astype(jnp.float32)

    m = jnp.max(x, axis=0)
    t = x - m[None]
    e = jnp.exp(t)
    s = jnp.sum(e, axis=0)
    logs = jnp.log(s)
    inv = 1.0 / s

    kio = jax.lax.broadcasted_iota(jnp.int32, (9, 81, _BC), 0)
    is_t = kio == tgt[None]
    t_sel = jnp.sum(jnp.where(is_t, t, 0.0), axis=0)
    e_sel = jnp.sum(jnp.where(is_t, e, 0.0), axis=0)
    pt = e_sel * inv
    ce = logs - t_sel
    q = 1.0 - pt
    focal_sum = jnp.sum(q * q * ce * mask)
    msum = jnp.sum(mask)

    ent = logs - inv * jnp.sum(e * t, axis=0)
    ent_sum = jnp.sum(ent * mask)

    p = e * inv[None]
    m1 = jnp.max(p, axis=0)
    eq = p == m1[None]
    am = jnp.min(jnp.where(eq, kio, 9), axis=0)
    m2 = jnp.max(jnp.where(kio == am[None], -1.0, p), axis=0)
    gap_sum = jnp.sum(jnp.maximum(1.0 - (m1 - m2), 0.0))

    mp = p * mask[None]
    sel = s_ref[...]
    cons_sq = jnp.float32(0.0)
    for k in range(9):
        sums_k = jax.lax.dot_general(
            sel, mp[k], (((1,), (0,)), ((), ())),
            preferred_element_type=jnp.float32)
        d = sums_k - 1.0
        cons_sq = cons_sq + jnp.sum(d * d)

    out_ref[0, 0, 0] = focal_sum
    out_ref[0, 0, 1] = msum
    out_ref[0, 0, 2] = cons_sq
    out_ref[0, 0, 3] = ent_sum
    out_ref[0, 0, 4] = gap_sum


def kernel(logits, targets, puzzles):
    b = logits.shape[0]
    nb = b // _BC
    lt = jnp.transpose(logits.reshape(b, 81, 9), (2, 1, 0))
    tg = targets.reshape(b, 81).astype(jnp.int32).T
    pz = puzzles.reshape(b, 81).astype(jnp.int32).T
    sel = _build_sel()

    partials = pl.pallas_call(
        _body,
        grid=(nb,),
        in_specs=[
            pl.BlockSpec((9, 81, _BC), lambda i: (0, 0, i)),
            pl.BlockSpec((81, _BC), lambda i: (0, i)),
            pl.BlockSpec((81, _BC), lambda i: (0, i)),
            pl.BlockSpec((27, 81), lambda i: (0, 0)),
        ],
        out_specs=pl.BlockSpec((1, 1, 8), lambda i: (i, 0, 0),
                               memory_space=pltpu.SMEM),
        out_shape=jax.ShapeDtypeStruct((nb, 1, 8), jnp.float32),
        compiler_params=pltpu.CompilerParams(
            dimension_semantics=("parallel",)),
    )(lt, tg, pz, sel)

    f = partials[:, 0, :5].sum(axis=0)
    cells = jnp.float32(b * 81)
    ce_loss = f[0] / (f[1] + _EPS)
    cons = f[2] / cells
    ent_loss = 0.1 * f[3] / (f[1] + _EPS)
    uniq_loss = 0.1 * f[4] / cells
    constraint = (cons + ent_loss + uniq_loss) * 0.2
    return ce_loss + _CONSTRAINT_WEIGHT * constraint
